# baseline (speedup 1.0000x reference)
import functools

import jax
import jax.numpy as jnp
from jax import lax
from jax.experimental import pallas as pl
from jax.experimental.pallas import tpu as pltpu
from jax.experimental.pallas import tpu_sc as plsc

NW = 32
CH = 128


def kernel(x, token_table, pos_table):
    B, S = x.shape
    V, D = token_table.shape
    N = B * S
    per_w = N // NW
    nch = per_w // CH
    idx = x.reshape(NW, nch, CH).astype(jnp.int32)
    mesh = plsc.VectorSubcoreMesh(core_axis_name="c", subcore_axis_name="s")

    @functools.partial(
        pl.kernel,
        mesh=mesh,
        out_type=jax.ShapeDtypeStruct((N, D), jnp.float32),
        scratch_types=[
            pltpu.VMEM((nch, CH), jnp.int32),
            pltpu.VMEM((CH, D), jnp.float32),
            pltpu.VMEM((CH, D), jnp.float32),
            pltpu.SemaphoreType.DMA,
        ],
        compiler_params=pltpu.CompilerParams(use_tc_tiling_on_sc=False),
    )
    def run(x_hbm, tok_hbm, pos_hbm, out_hbm, idx_v, tok_v, pos_v, sem):
        cid = lax.axis_index("c")
        sid = lax.axis_index("s")
        wid = sid * 2 + cid
        base = wid * per_w
        s_base = lax.rem(base, S)
        pltpu.sync_copy(x_hbm.at[wid], idx_v)

        def chunk(c, carry):
            g = pltpu.async_copy(tok_hbm.at[idx_v.at[c]], tok_v, sem)
            pltpu.sync_copy(pos_hbm.at[pl.ds(s_base + c * CH, CH)], pos_v)
            g.wait()

            def row(i, carry2):
                for j in range(D // 16):
                    sl = pl.ds(j * 16, 16)
                    tok_v[i, sl] = tok_v[i, sl] + pos_v[i, sl]
                return carry2

            lax.fori_loop(0, CH, row, 0)
            pltpu.sync_copy(tok_v, out_hbm.at[pl.ds(base + c * CH, CH)])
            return carry

        lax.fori_loop(0, nch, chunk, 0)

    out = run(idx, token_table, pos_table)
    return out.reshape(B, S, D)


# speedup vs baseline: 1.5660x; 1.5660x over previous
import functools

import jax
import jax.numpy as jnp
from jax import lax
from jax.experimental import pallas as pl
from jax.experimental.pallas import tpu as pltpu
from jax.experimental.pallas import tpu_sc as plsc

NW = 32
CH = 128


def kernel(x, token_table, pos_table):
    B, S = x.shape
    V, D = token_table.shape
    N = B * S
    per_w = N // NW
    nch = per_w // CH
    xf = x.reshape(NW, nch, CH).astype(jnp.int32)
    mesh = plsc.VectorSubcoreMesh(core_axis_name="c", subcore_axis_name="s")

    @functools.partial(
        pl.kernel,
        mesh=mesh,
        out_type=jax.ShapeDtypeStruct((N, D), jnp.float32),
        scratch_types=[
            pltpu.VMEM((nch, CH), jnp.int32),
            pltpu.VMEM((CH, D), jnp.float32),
            pltpu.VMEM((CH, D), jnp.float32),
            pltpu.SemaphoreType.DMA,
            pltpu.SemaphoreType.DMA,
        ],
        compiler_params=pltpu.CompilerParams(needs_layout_passes=False),
    )
    def run(x_hbm, tok_hbm, pos_hbm, out_hbm, idx_v, tok_v, out_v, gsem, psem):
        cid = lax.axis_index("c")
        sid = lax.axis_index("s")
        wid = sid * 2 + cid
        base = wid * per_w
        s_base = lax.rem(base, S)
        pltpu.sync_copy(x_hbm.at[wid], idx_v)
        lanes = lax.iota(jnp.int32, 16)

        def chunk(c, carry):
            p = pltpu.async_copy(
                pos_hbm.at[pl.ds(s_base + c * CH, CH)], out_v, psem)
            for g in range(CH // 16):
                xv = idx_v[c, pl.ds(g * 16, 16)]
                for l in range(16):
                    v = jnp.max(jnp.where(lanes == l, xv, 0))
                    pltpu.async_copy(
                        tok_hbm.at[pl.ds(v, 1)],
                        tok_v.at[pl.ds(g * 16 + l, 1)], gsem)
            p.wait()
            drain = pltpu.make_async_copy(
                tok_hbm.at[pl.ds(0, 1)], tok_v.at[pl.ds(0, 1)], gsem)
            for r in range(CH):
                drain.wait()
            for r in range(CH):
                for j in range(D // 16):
                    sl = pl.ds(j * 16, 16)
                    out_v[r, sl] = out_v[r, sl] + tok_v[r, sl]
            pltpu.sync_copy(out_v, out_hbm.at[pl.ds(base + c * CH, CH)])
            return carry

        lax.fori_loop(0, nch, chunk, 0)

    out = run(xf, token_table, pos_table)
    return out.reshape(B, S, D)
